# initial kernel scaffold (unmeasured)
import jax
import jax.numpy as jnp
from jax import lax
from jax.experimental import pallas as pl
from jax.experimental.pallas import tpu as pltpu

N_DEV = 4


def kernel(x, dy):
    k, m = x.shape
    _, n = dy.shape
    m_out = m // N_DEV

    x = x.astype(jnp.bfloat16)
    dy = dy.astype(jnp.bfloat16)

    def body(x_ref, dy_ref, out_ref, comm_ref, send_sems, recv_sems):
        d = lax.axis_index("i")
        left = (d - 1) % N_DEV
        right = (d + 1) % N_DEV

        barrier_sem = pltpu.get_barrier_semaphore()
        for nbr in (left, right):
            pl.semaphore_signal(
                barrier_sem, inc=1,
                device_id=(nbr,), device_id_type=pl.DeviceIdType.MESH,
            )
        pl.semaphore_wait(barrier_sem, 2)

        def partial(c):
            xs = x_ref[:, pl.ds(c * m_out, m_out)]
            return lax.dot_general(
                xs, dy_ref[...],
                dimension_numbers=(((0,), (0,)), ((), ())),
                preferred_element_type=jnp.float32,
            )

        comm_ref[0, :, :] = partial((d - 1) % N_DEV).astype(jnp.bfloat16)

        for h in range(N_DEV - 1):
            send_slot = h % 2
            recv_slot = (h + 1) % 2
            rdma = pltpu.make_async_remote_copy(
                src_ref=comm_ref.at[send_slot],
                dst_ref=comm_ref.at[recv_slot],
                send_sem=send_sems.at[send_slot],
                recv_sem=recv_sems.at[recv_slot],
                device_id=(right,),
                device_id_type=pl.DeviceIdType.MESH,
            )
            rdma.start()
            rdma.wait()

            c = (d - 2 - h) % N_DEV
            acc = partial(c) + comm_ref[recv_slot, :, :].astype(jnp.float32)
            if h < N_DEV - 2:
                comm_ref[recv_slot, :, :] = acc.astype(jnp.bfloat16)
            else:
                out_ref[...] = acc

    return pl.pallas_call(
        body,
        out_shape=jax.ShapeDtypeStruct((m_out, n), jnp.float32),
        in_specs=[
            pl.BlockSpec(memory_space=pltpu.VMEM),
            pl.BlockSpec(memory_space=pltpu.VMEM),
        ],
        out_specs=pl.BlockSpec(memory_space=pltpu.VMEM),
        scratch_shapes=[
            pltpu.VMEM((2, m_out, n), jnp.bfloat16),
            pltpu.SemaphoreType.DMA((2,)),
            pltpu.SemaphoreType.DMA((2,)),
        ],
        compiler_params=pltpu.CompilerParams(collective_id=0),
    )(x, dy)


# baseline (device time: 440622 ns/iter reference)
import jax
import jax.numpy as jnp
from jax import lax
from jax.experimental import pallas as pl
from jax.experimental.pallas import tpu as pltpu

N_DEV = 4
TN = 512


def kernel(x, dy):
    k, m = x.shape
    _, n = dy.shape
    m_out = m // N_DEV

    d = lax.axis_index("i")
    x = x.astype(jnp.bfloat16)
    dy = dy.astype(jnp.bfloat16)
    x = jnp.roll(x, -(d + 1) * m_out, axis=1)

    def body(x_ref, dy_hbm, out_ref, comm_ref, dy_tile, dma_sems,
             send_sems, recv_sems):
        my_d = lax.axis_index("i")
        dst = (my_d - 1) % N_DEV
        src = (my_d + 1) % N_DEV

        barrier_sem = pltpu.get_barrier_semaphore()
        for nbr in (dst, src):
            pl.semaphore_signal(
                barrier_sem, inc=1,
                device_id=(nbr,), device_id_type=pl.DeviceIdType.MESH,
            )
        pl.semaphore_wait(barrier_sem, 2)

        n_tiles = n // TN

        def load_tile(t, slot):
            cp = pltpu.make_async_copy(
                dy_hbm.at[:, pl.ds(t * TN, TN)],
                dy_tile.at[slot],
                dma_sems.at[slot],
            )
            cp.start()
            return cp

        def compute_chunk(j, combine):
            xs = x_ref[:, j * m_out:(j + 1) * m_out]
            cps = {0: load_tile(0, 0)}
            for t in range(n_tiles):
                if t + 1 < n_tiles:
                    cps[t + 1] = load_tile(t + 1, (t + 1) % 2)
                cps[t].wait()
                p = lax.dot_general(
                    xs, dy_tile[t % 2],
                    dimension_numbers=(((0,), (0,)), ((), ())),
                    preferred_element_type=jnp.float32,
                )
                combine(t, p)

        def seed(t, p):
            comm_ref[0, :, pl.ds(t * TN, TN)] = p.astype(jnp.bfloat16)
        compute_chunk(0, seed)

        for h in range(N_DEV - 1):
            send_slot = h % 2
            recv_slot = (h + 1) % 2
            rdma = pltpu.make_async_remote_copy(
                src_ref=comm_ref.at[send_slot],
                dst_ref=comm_ref.at[recv_slot],
                send_sem=send_sems.at[send_slot],
                recv_sem=recv_sems.at[recv_slot],
                device_id=(dst,),
                device_id_type=pl.DeviceIdType.MESH,
            )
            rdma.start()
            rdma.wait()

            j = h + 1
            if h < N_DEV - 2:
                def acc(t, p, _slot=recv_slot):
                    sl = pl.ds(t * TN, TN)
                    comm_ref[_slot, :, sl] = (
                        p + comm_ref[_slot, :, sl].astype(jnp.float32)
                    ).astype(jnp.bfloat16)
            else:
                def acc(t, p, _slot=recv_slot):
                    sl = pl.ds(t * TN, TN)
                    out_ref[:, sl] = p + comm_ref[_slot, :, sl].astype(
                        jnp.float32)
            compute_chunk(j, acc)

    return pl.pallas_call(
        body,
        out_shape=jax.ShapeDtypeStruct((m_out, n), jnp.float32),
        in_specs=[
            pl.BlockSpec(memory_space=pltpu.VMEM),
            pl.BlockSpec(memory_space=pl.ANY),
        ],
        out_specs=pl.BlockSpec(memory_space=pltpu.VMEM),
        scratch_shapes=[
            pltpu.VMEM((2, m_out, n), jnp.bfloat16),
            pltpu.VMEM((2, k, TN), jnp.bfloat16),
            pltpu.SemaphoreType.DMA((2,)),
            pltpu.SemaphoreType.DMA((2,)),
            pltpu.SemaphoreType.DMA((2,)),
        ],
        compiler_params=pltpu.CompilerParams(
            collective_id=0,
            vmem_limit_bytes=40 * 1024 * 1024,
        ),
    )(x, dy)


# device time: 246775 ns/iter; 1.7855x vs baseline; 1.7855x over previous
import jax
import jax.numpy as jnp
from jax import lax
from jax.experimental import pallas as pl
from jax.experimental.pallas import tpu as pltpu

N_DEV = 4
TN = 512


def kernel(x, dy):
    k, m = x.shape
    _, n = dy.shape
    m_out = m // N_DEV

    d = lax.axis_index("i")
    x = x.astype(jnp.bfloat16)
    dy = dy.astype(jnp.bfloat16)
    x = jnp.roll(x, -(d + 1) * m_out, axis=1)

    def body(x_ref, dy_hbm, out_hbm, commA, commB, dy_tile, ostage,
             dma_sems, o_sems, sendA, recvA, sendB, recvB):
        my_d = lax.axis_index("i")
        left = (my_d - 1) % N_DEV
        right = (my_d + 1) % N_DEV

        barrier_sem = pltpu.get_barrier_semaphore()
        for nbr in (left, right):
            pl.semaphore_signal(
                barrier_sem, inc=1,
                device_id=(nbr,), device_id_type=pl.DeviceIdType.MESH,
            )
        pl.semaphore_wait(barrier_sem, 2)

        n_tiles = n // TN
        half = n_tiles // 2

        def load_tile(t, slot):
            cp = pltpu.make_async_copy(
                dy_hbm.at[:, pl.ds(t * TN, TN)],
                dy_tile.at[slot],
                dma_sems.at[slot],
            )
            cp.start()
            return cp

        def xs(b):
            return x_ref[:, b * m_out:(b + 1) * m_out]

        def compute_phase(bA, bB, combA, combB, cps):
            for t in range(n_tiles):
                if t + 1 < n_tiles and (t + 1) not in cps:
                    cps[t + 1] = load_tile(t + 1, (t + 1) % 2)
                cps[t].wait()
                b = bA if t < half else bB
                p = lax.dot_general(
                    xs(b), dy_tile[t % 2],
                    dimension_numbers=(((0,), (0,)), ((), ())),
                    preferred_element_type=jnp.float32,
                )
                if t < half:
                    combA(t, p)
                else:
                    combB(t - half, p)

        def write_slot(ref, slot):
            def comb(t, p):
                ref[slot, :, pl.ds(t * TN, TN)] = p.astype(jnp.bfloat16)
            return comb

        def rdma(ref, s_slot, r_slot, s_sems, r_sems, dev):
            return pltpu.make_async_remote_copy(
                src_ref=ref.at[s_slot],
                dst_ref=ref.at[r_slot],
                send_sem=s_sems.at[s_slot],
                recv_sem=r_sems.at[r_slot],
                device_id=(dev,),
                device_id_type=pl.DeviceIdType.MESH,
            )

        def accum(ref, dst_slot):
            ref[dst_slot, :, :] = (
                ref[dst_slot, :, :].astype(jnp.float32)
                + ref[2, :, :].astype(jnp.float32)
            ).astype(jnp.bfloat16)

        cps = {0: load_tile(0, 0)}
        compute_phase(0, 2, write_slot(commA, 0), write_slot(commB, 0), cps)

        ra = rdma(commA, 0, 1, sendA, recvA, left)
        rb = rdma(commB, 0, 1, sendB, recvB, right)
        ra.start()
        rb.start()
        cps = {0: load_tile(0, 0), 1: load_tile(1, 1)}
        compute_phase(1, 1, write_slot(commA, 2), write_slot(commB, 2), cps)
        ra.wait_recv()
        rb.wait_recv()
        accum(commA, 1)
        accum(commB, 1)
        ra.wait_send()
        rb.wait_send()

        ra = rdma(commA, 1, 0, sendA, recvA, left)
        rb = rdma(commB, 1, 0, sendB, recvB, right)
        ra.start()
        rb.start()
        cps = {0: load_tile(0, 0), 1: load_tile(1, 1)}
        compute_phase(2, 0, write_slot(commA, 2), write_slot(commB, 2), cps)
        ra.wait_recv()
        rb.wait_recv()
        accum(commA, 0)
        accum(commB, 0)
        ra.wait_send()
        rb.wait_send()

        ra = rdma(commA, 0, 1, sendA, recvA, left)
        rb = rdma(commB, 0, 1, sendB, recvB, right)
        ra.start()
        rb.start()
        cps = {0: load_tile(0, 0), 1: load_tile(1, 1)}
        compute_phase(3, 3, write_slot(commA, 2), write_slot(commB, 2), cps)
        ra.wait_recv()
        rb.wait_recv()

        ocps = {}
        for t in range(n_tiles):
            slot = t % 2
            if t >= 2:
                ocps[t - 2].wait()
            ref = commA if t < half else commB
            tt = t if t < half else t - half
            sl = pl.ds(tt * TN, TN)
            ostage[slot, :, :] = (
                ref[1, :, sl].astype(jnp.float32)
                + ref[2, :, sl].astype(jnp.float32)
            )
            cp = pltpu.make_async_copy(
                ostage.at[slot],
                out_hbm.at[:, pl.ds(t * TN, TN)],
                o_sems.at[slot],
            )
            cp.start()
            ocps[t] = cp
        ocps[n_tiles - 2].wait()
        ocps[n_tiles - 1].wait()
        ra.wait_send()
        rb.wait_send()

    return pl.pallas_call(
        body,
        out_shape=jax.ShapeDtypeStruct((m_out, n), jnp.float32),
        in_specs=[
            pl.BlockSpec(memory_space=pltpu.VMEM),
            pl.BlockSpec(memory_space=pl.ANY),
        ],
        out_specs=pl.BlockSpec(memory_space=pl.ANY),
        scratch_shapes=[
            pltpu.VMEM((3, m_out, n // 2), jnp.bfloat16),
            pltpu.VMEM((3, m_out, n // 2), jnp.bfloat16),
            pltpu.VMEM((2, k, TN), jnp.bfloat16),
            pltpu.VMEM((2, m_out, TN), jnp.float32),
            pltpu.SemaphoreType.DMA((2,)),
            pltpu.SemaphoreType.DMA((2,)),
            pltpu.SemaphoreType.DMA((3,)),
            pltpu.SemaphoreType.DMA((3,)),
            pltpu.SemaphoreType.DMA((3,)),
            pltpu.SemaphoreType.DMA((3,)),
        ],
        compiler_params=pltpu.CompilerParams(
            collective_id=0,
            vmem_limit_bytes=54 * 1024 * 1024,
        ),
    )(x, dy)


# device time: 208626 ns/iter; 2.1120x vs baseline; 1.1829x over previous
import jax
import jax.numpy as jnp
from jax import lax
from jax.experimental import pallas as pl
from jax.experimental.pallas import tpu as pltpu

N_DEV = 4
TN = 512
N_LANES = 4

BLK_LEFT = (0, 1, 2, 3)
BLK_RIGHT = (2, 1, 0, 3)


def kernel(x, dy):
    k, m = x.shape
    _, n = dy.shape
    m_out = m // N_DEV
    ncols = n // N_LANES

    d = lax.axis_index("i")
    x = x.astype(jnp.bfloat16)
    x = jnp.roll(x, -(d + 1) * m_out, axis=1)

    def body(x_ref, dy_hbm, out_hbm, dybf, comm, dyf_tile, dy_tile, ostage,
             f_sems, c_sems, o_sems, send_sems, recv_sems):
        my_d = lax.axis_index("i")
        left = (my_d - 1) % N_DEV
        right = (my_d + 1) % N_DEV

        barrier_sem = pltpu.get_barrier_semaphore()
        for nbr in (left, right):
            pl.semaphore_signal(
                barrier_sem, inc=1,
                device_id=(nbr,), device_id_type=pl.DeviceIdType.MESH,
            )
        pl.semaphore_wait(barrier_sem, 2)

        n_tiles = n // TN
        tps = ncols // TN

        def xs(b):
            return x_ref[:, b * m_out:(b + 1) * m_out]

        def rdma(r, s_slot, r_slot, dev):
            return pltpu.make_async_remote_copy(
                src_ref=comm.at[r, s_slot],
                dst_ref=comm.at[r, r_slot],
                send_sem=send_sems.at[r, s_slot],
                recv_sem=recv_sems.at[r, r_slot],
                device_id=(dev,),
                device_id_type=pl.DeviceIdType.MESH,
            )

        rds = {}

        def start_hop(r, h):
            dev = left if r < N_LANES // 2 else right
            rd = rdma(r, h % 2, (h + 1) % 2, dev)
            rd.start()
            rds[(r, h)] = rd

        def load_f32(t, slot):
            cp = pltpu.make_async_copy(
                dy_hbm.at[:, pl.ds(t * TN, TN)], dyf_tile.at[slot],
                f_sems.at[slot])
            cp.start()
            return cp

        fcps = {0: load_f32(0, 0), 1: load_f32(1, 1)}
        wbs = {}
        for t in range(n_tiles):
            r, ti = divmod(t, tps)
            if t + 1 < n_tiles and (t + 1) not in fcps:
                fcps[t + 1] = load_f32(t + 1, (t + 1) % 2)
            fcps[t].wait()
            if t >= 2:
                wbs[t - 2].wait()
            dy_tile[t % 2, :, :] = dyf_tile[t % 2, :, :].astype(jnp.bfloat16)
            wb = pltpu.make_async_copy(
                dy_tile.at[t % 2], dybf.at[:, pl.ds(t * TN, TN)],
                c_sems.at[t % 2])
            wb.start()
            wbs[t] = wb
            blk = BLK_LEFT[0] if r < N_LANES // 2 else BLK_RIGHT[0]
            p = lax.dot_general(
                xs(blk), dy_tile[t % 2],
                dimension_numbers=(((0,), (0,)), ((), ())),
                preferred_element_type=jnp.float32,
            )
            comm[r, 0, :, pl.ds(ti * TN, TN)] = p.astype(jnp.bfloat16)
            if ti == tps - 1:
                start_hop(r, 0)
        wbs[n_tiles - 2].wait()
        wbs[n_tiles - 1].wait()

        def load_bf(t, slot):
            cp = pltpu.make_async_copy(
                dybf.at[:, pl.ds(t * TN, TN)], dy_tile.at[slot],
                f_sems.at[slot])
            cp.start()
            return cp

        ocps = {}
        for ph in range(1, N_DEV):
            cps = {0: load_bf(0, 0), 1: load_bf(1, 1)}
            for t in range(n_tiles):
                r, ti = divmod(t, tps)
                if t + 1 < n_tiles and (t + 1) not in cps:
                    cps[t + 1] = load_bf(t + 1, (t + 1) % 2)
                if ti == 0:
                    rds[(r, ph - 1)].wait_recv()
                    rds[(r, ph - 1)].wait_send()
                cps[t].wait()
                blk = (BLK_LEFT[ph] if r < N_LANES // 2 else BLK_RIGHT[ph])
                p = lax.dot_general(
                    xs(blk), dy_tile[t % 2],
                    dimension_numbers=(((0,), (0,)), ((), ())),
                    preferred_element_type=jnp.float32,
                )
                rslot = ph % 2
                sl = pl.ds(ti * TN, TN)
                if ph < N_DEV - 1:
                    comm[r, rslot, :, sl] = (
                        p + comm[r, rslot, :, sl].astype(jnp.float32)
                    ).astype(jnp.bfloat16)
                else:
                    if t >= 2:
                        ocps[t - 2].wait()
                    ostage[t % 2, :, :] = (
                        p + comm[r, rslot, :, sl].astype(jnp.float32))
                    ocp = pltpu.make_async_copy(
                        ostage.at[t % 2], out_hbm.at[:, pl.ds(t * TN, TN)],
                        o_sems.at[t % 2])
                    ocp.start()
                    ocps[t] = ocp
                if ti == tps - 1 and ph < N_DEV - 1:
                    start_hop(r, ph)
        ocps[n_tiles - 2].wait()
        ocps[n_tiles - 1].wait()

    out, _ = pl.pallas_call(
        body,
        out_shape=[
            jax.ShapeDtypeStruct((m_out, n), jnp.float32),
            jax.ShapeDtypeStruct((k, n), jnp.bfloat16),
        ],
        in_specs=[
            pl.BlockSpec(memory_space=pltpu.VMEM),
            pl.BlockSpec(memory_space=pl.ANY),
        ],
        out_specs=[
            pl.BlockSpec(memory_space=pl.ANY),
            pl.BlockSpec(memory_space=pl.ANY),
        ],
        scratch_shapes=[
            pltpu.VMEM((N_LANES, 2, m_out, ncols), jnp.bfloat16),
            pltpu.VMEM((2, k, TN), jnp.float32),
            pltpu.VMEM((2, k, TN), jnp.bfloat16),
            pltpu.VMEM((2, m_out, TN), jnp.float32),
            pltpu.SemaphoreType.DMA((2,)),
            pltpu.SemaphoreType.DMA((2,)),
            pltpu.SemaphoreType.DMA((2,)),
            pltpu.SemaphoreType.DMA((N_LANES, 2)),
            pltpu.SemaphoreType.DMA((N_LANES, 2)),
        ],
        compiler_params=pltpu.CompilerParams(
            collective_id=0,
            vmem_limit_bytes=48 * 1024 * 1024,
        ),
    )(x, dy)
    return out


# device time: 183833 ns/iter; 2.3969x vs baseline; 1.1349x over previous
import jax
import jax.numpy as jnp
from jax import lax
from jax.experimental import pallas as pl
from jax.experimental.pallas import tpu as pltpu

N_DEV = 4
TN = 512
N_LANES = 16
NSLOT = 6

BLK_LEFT = (0, 1, 2, 3)
BLK_RIGHT = (2, 1, 0, 3)


def kernel(x, dy):
    k, m = x.shape
    _, n = dy.shape
    m_out = m // N_DEV
    ncols = n // N_LANES

    def body(x_hbm, dy_hbm, out_hbm, dybf, xbf, xstage, comm, dyf_tile, dy_tile, ostage,
             x_sems, f_sems, c_sems, o_sems, send_sems, recv_sems):
        my_d = lax.axis_index("i")
        left = (my_d - 1) % N_DEV
        right = (my_d + 1) % N_DEV

        barrier_sem = pltpu.get_barrier_semaphore()
        for nbr in (left, right):
            pl.semaphore_signal(
                barrier_sem, inc=1,
                device_id=(nbr,), device_id_type=pl.DeviceIdType.MESH,
            )
        pl.semaphore_wait(barrier_sem, 2)

        n_tiles = n // TN
        tps = ncols // TN

        def load_x(j, slot):
            srcb = (my_d + 1 + j) % N_DEV
            cp = pltpu.make_async_copy(
                x_hbm.at[:, pl.ds(srcb * m_out, m_out)], xstage.at[slot],
                x_sems.at[slot])
            cp.start()
            return cp

        xcps = {0: load_x(0, 0), 2: load_x(2, 1)}
        for j, slot in ((0, 0), (2, 1)):
            xcps[j].wait()
            xbf[:, j * m_out:(j + 1) * m_out] = (
                xstage[slot, :, :].astype(jnp.bfloat16))

        def xs(b):
            return xbf[:, b * m_out:(b + 1) * m_out]

        def dot(b, t):
            return lax.dot_general(
                xs(b), dy_tile[t % NSLOT],
                dimension_numbers=(((0,), (0,)), ((), ())),
                preferred_element_type=jnp.float32,
            )

        def rdma(r, s_slot, r_slot, dev):
            return pltpu.make_async_remote_copy(
                src_ref=comm.at[r, s_slot],
                dst_ref=comm.at[r, r_slot],
                send_sem=send_sems.at[r, s_slot],
                recv_sem=recv_sems.at[r, r_slot],
                device_id=(dev,),
                device_id_type=pl.DeviceIdType.MESH,
            )

        rds = {}

        def start_hop(r, h):
            dev = left if r < N_LANES // 2 else right
            rd = rdma(r, h % 2, (h + 1) % 2, dev)
            rd.start()
            rds[(r, h)] = rd

        def load_f32(t, slot):
            cp = pltpu.make_async_copy(
                dy_hbm.at[:, pl.ds(t * TN, TN)], dyf_tile.at[slot],
                f_sems.at[slot])
            cp.start()
            return cp

        fcps = {0: load_f32(0, 0), 1: load_f32(1, 1)}
        wbs = {}
        for t in range(n_tiles):
            r, ti = divmod(t, tps)
            if t + 1 < n_tiles and (t + 1) not in fcps:
                fcps[t + 1] = load_f32(t + 1, (t + 1) % 2)
            fcps[t].wait()
            if t >= NSLOT:
                wbs[t - NSLOT].wait()
            dy_tile[t % NSLOT, :, :] = (
                dyf_tile[t % 2, :, :].astype(jnp.bfloat16))
            wb = pltpu.make_async_copy(
                dy_tile.at[t % NSLOT], dybf.at[:, pl.ds(t * TN, TN)],
                c_sems.at[t % NSLOT])
            wb.start()
            wbs[t] = wb
            blk = BLK_LEFT[0] if r < N_LANES // 2 else BLK_RIGHT[0]
            p = dot(blk, t)
            comm[r, 0, :, pl.ds(ti * TN, TN)] = p.astype(jnp.bfloat16)
            if ti == tps - 1:
                start_hop(r, 0)
            if t == 0:
                xcps[1] = load_x(1, 0)
                xcps[3] = load_x(3, 1)
        for j, slot in ((1, 0), (3, 1)):
            xcps[j].wait()
            xbf[:, j * m_out:(j + 1) * m_out] = (
                xstage[slot, :, :].astype(jnp.bfloat16))
        for t in range(n_tiles - NSLOT, n_tiles):
            wbs[t].wait()

        def load_bf(t):
            cp = pltpu.make_async_copy(
                dybf.at[:, pl.ds(t * TN, TN)], dy_tile.at[t % NSLOT],
                f_sems.at[t % NSLOT])
            cp.start()
            return cp

        ocps = {}
        for ph in range(1, N_DEV):
            cps = {t: load_bf(t) for t in range(NSLOT - 1)}
            for t in range(n_tiles):
                r, ti = divmod(t, tps)
                for tn in range(t + 1, min(t + NSLOT, n_tiles)):
                    if tn not in cps:
                        cps[tn] = load_bf(tn)
                if ti == 0:
                    rds[(r, ph - 1)].wait_recv()
                    rds[(r, ph - 1)].wait_send()
                cps[t].wait()
                blk = (BLK_LEFT[ph] if r < N_LANES // 2 else BLK_RIGHT[ph])
                p = dot(blk, t)
                rslot = ph % 2
                sl = pl.ds(ti * TN, TN)
                if ph < N_DEV - 1:
                    comm[r, rslot, :, sl] = (
                        p + comm[r, rslot, :, sl].astype(jnp.float32)
                    ).astype(jnp.bfloat16)
                else:
                    if t >= 2:
                        ocps[t - 2].wait()
                    ostage[t % 2, :, :] = (
                        p + comm[r, rslot, :, sl].astype(jnp.float32))
                    ocp = pltpu.make_async_copy(
                        ostage.at[t % 2], out_hbm.at[:, pl.ds(t * TN, TN)],
                        o_sems.at[t % 2])
                    ocp.start()
                    ocps[t] = ocp
                if ti == tps - 1 and ph < N_DEV - 1:
                    start_hop(r, ph)
        ocps[n_tiles - 2].wait()
        ocps[n_tiles - 1].wait()

    out, _ = pl.pallas_call(
        body,
        out_shape=[
            jax.ShapeDtypeStruct((m_out, n), jnp.float32),
            jax.ShapeDtypeStruct((k, n), jnp.bfloat16),
        ],
        in_specs=[
            pl.BlockSpec(memory_space=pl.ANY),
            pl.BlockSpec(memory_space=pl.ANY),
        ],
        out_specs=[
            pl.BlockSpec(memory_space=pl.ANY),
            pl.BlockSpec(memory_space=pl.ANY),
        ],
        scratch_shapes=[
            pltpu.VMEM((k, m), jnp.bfloat16),
            pltpu.VMEM((2, k, m_out), jnp.float32),
            pltpu.VMEM((N_LANES, 2, m_out, ncols), jnp.bfloat16),
            pltpu.VMEM((2, k, TN), jnp.float32),
            pltpu.VMEM((NSLOT, k, TN), jnp.bfloat16),
            pltpu.VMEM((2, m_out, TN), jnp.float32),
            pltpu.SemaphoreType.DMA((2,)),
            pltpu.SemaphoreType.DMA((NSLOT,)),
            pltpu.SemaphoreType.DMA((NSLOT,)),
            pltpu.SemaphoreType.DMA((2,)),
            pltpu.SemaphoreType.DMA((N_LANES, 2)),
            pltpu.SemaphoreType.DMA((N_LANES, 2)),
        ],
        compiler_params=pltpu.CompilerParams(
            collective_id=0,
            vmem_limit_bytes=63 * 1024 * 1024,
        ),
    )(x, dy)
    return out
